# baseline (speedup 1.0000x reference)
import functools

import jax
import jax.numpy as jnp
from jax import lax
from jax.experimental import pallas as pl
from jax.experimental.pallas import tpu as pltpu
from jax.experimental.pallas import tpu_sc as plsc

F32 = jnp.float32
I32 = jnp.int32

N_NODES = 4000
NP = 4096
SH = 4224
DUMMY = 4096
NFEAT = 128
HID = 128
CONV = 64
T = 7
P = 8000
PP = 8192
FLAT = PP * T
E = 128000
EP = 131072
L = 10000
LP = 10240
NW = 32

_mesh = lambda: plsc.VectorSubcoreMesh(core_axis_name="c", subcore_axis_name="s")


@functools.cache
def _mk_gather(D, R, rw, ring, label):
    depth = ring // 2

    @functools.partial(
        pl.kernel,
        out_type=jax.ShapeDtypeStruct((R * 128, D), F32),
        mesh=_mesh(),
        scratch_types=[
            pltpu.VMEM((rw, 128), I32),
            pltpu.VMEM((ring * 128, D), F32),
            pltpu.SemaphoreType.DMA,
            pltpu.SemaphoreType.DMA,
        ],
        name=label,
    )
    def k(table, idx, out, idx_v, rows_v, gsem, wsem):
        c = lax.axis_index("c")
        s = lax.axis_index("s")
        wid = s * 2 + c
        base = wid * rw
        pltpu.sync_copy(idx.at[wid], idx_v)

        def buf(j):
            return rows_v.at[pl.ds((j % ring) * 128, 128)]

        gd = {}
        wd = {}
        for j in range(min(depth, rw)):
            gd[j] = pltpu.async_copy(table.at[idx_v.at[j]], buf(j), gsem)
        for j in range(rw):
            gd[j].wait()
            wd[j] = pltpu.async_copy(buf(j), out.at[pl.ds((base + j) * 128, 128)], wsem)
            if j >= depth:
                wd[j - depth].wait()
            if j + depth < rw:
                gd[j + depth] = pltpu.async_copy(
                    table.at[idx_v.at[j + depth]], buf(j + depth), gsem)
        for j in range(max(rw - depth, 0), rw):
            wd[j].wait()

    return k


def _gather_paths(table, idx):
    return _mk_gather(NFEAT, FLAT // 128, FLAT // 128 // NW, 4, "path_gather")(table, idx)


def _gather_links(table, idx):
    return _mk_gather(256, (2 * LP) // 128, (2 * LP) // 128 // NW, 2, "link_gather")(table, idx)


_SROWS = SH // 16


@functools.cache
def _mk_edge_scatter():
    @functools.partial(
        pl.kernel,
        out_type=jax.ShapeDtypeStruct((2, SH, CONV), F32),
        mesh=_mesh(),
        scratch_types=[
            pltpu.VMEM((EP // 128 // NW, 128), I32),
            pltpu.VMEM((EP // 128 // NW, 128), I32),
            pltpu.VMEM((8 * 128, CONV), F32),
            pltpu.VMEM_SHARED((SH, CONV), F32),
            pltpu.SemaphoreType.DMA,
            pltpu.SemaphoreType.DMA,
        ],
        compiler_params=pltpu.CompilerParams(use_tc_tiling_on_sc=False),
        name="edge_scatter",
    )
    def k(xw, sI, dI, z, out, sv, dv, rows, shared, gsem, ssem):
        c = lax.axis_index("c")
        s = lax.axis_index("s")
        wid = s * 2 + c
        nchunk = EP // 128 // NW
        ring, depth = 8, 4
        pltpu.sync_copy(z.at[pl.ds(s * _SROWS, _SROWS)], shared.at[pl.ds(s * _SROWS, _SROWS)])
        pltpu.sync_copy(sI.at[wid], sv)
        pltpu.sync_copy(dI.at[wid], dv)
        plsc.subcore_barrier()

        def buf(j):
            return rows.at[pl.ds((j % ring) * 128, 128)]

        gd = {}
        sd = {}
        for j in range(depth):
            gd[j] = pltpu.async_copy(xw.at[sv.at[j]], buf(j), gsem)
        for j in range(nchunk):
            gd[j].wait()
            sd[j] = pltpu.async_copy(buf(j), shared.at[dv.at[j]], ssem, add=True)
            if j >= depth:
                sd[j - depth].wait()
            if j + depth < nchunk:
                gd[j + depth] = pltpu.async_copy(
                    xw.at[sv.at[j + depth]], buf(j + depth), gsem)
        for j in range(nchunk - depth, nchunk):
            sd[j].wait()
        plsc.subcore_barrier()
        pltpu.sync_copy(shared.at[pl.ds(s * _SROWS, _SROWS)], out.at[c, pl.ds(s * _SROWS, _SROWS)])

    return k


def _edge_scatter(xw, sI, dI, z):
    return _mk_edge_scatter()(xw, sI, dI, z)


@functools.cache
def _mk_deg_hist():
    @functools.partial(
        pl.kernel,
        out_type=jax.ShapeDtypeStruct((2, SH, 16), F32),
        mesh=_mesh(),
        scratch_types=[
            pltpu.VMEM((EP // 128 // NW, 128), I32),
            pltpu.VMEM((128, 16), F32),
            pltpu.VMEM_SHARED((SH, 16), F32),
            pltpu.SemaphoreType.DMA,
        ],
        compiler_params=pltpu.CompilerParams(use_tc_tiling_on_sc=False),
        name="deg_hist",
    )
    def k(dI, z, ones, out, dv, ones_v, shared, sem):
        c = lax.axis_index("c")
        s = lax.axis_index("s")
        wid = s * 2 + c
        nchunk = EP // 128 // NW
        pltpu.sync_copy(z.at[pl.ds(s * _SROWS, _SROWS)], shared.at[pl.ds(s * _SROWS, _SROWS)])
        pltpu.sync_copy(dI.at[wid], dv)
        pltpu.sync_copy(ones, ones_v)
        plsc.subcore_barrier()

        sd = {}
        for j in range(nchunk):
            sd[j] = pltpu.async_copy(ones_v, shared.at[dv.at[j]], sem, add=True)
            if j >= 8:
                sd[j - 8].wait()
        for j in range(nchunk - 8, nchunk):
            sd[j].wait()
        plsc.subcore_barrier()
        pltpu.sync_copy(shared.at[pl.ds(s * _SROWS, _SROWS)], out.at[c, pl.ds(s * _SROWS, _SROWS)])

    return k


def _deg_hist(dI, z, ones):
    return _mk_deg_hist()(dI, z, ones)


_PB = 512
_NB = 256
_GRID = PP // _PB


def _lstm_body(f0, f1, f2, f3, f4, f5, f6, wih, whh, b, cw, c0, c1, a, out):
    fs = (f0, f1, f2, f3, f4, f5, f6)
    BF = jnp.bfloat16
    W_ih = wih[:].astype(BF)
    W_hh = whh[:].astype(BF)
    bb = b[:]
    h = jnp.zeros((_PB, HID), F32)
    c = jnp.zeros((_PB, HID), F32)
    for t in range(T):
        xt = fs[t][:].astype(BF)
        g = (jnp.dot(xt, W_ih, preferred_element_type=F32)
             + jnp.dot(h.astype(BF), W_hh, preferred_element_type=F32) + bb)
        ig = jax.nn.sigmoid(g[:, :HID])
        fg = jax.nn.sigmoid(g[:, HID:2 * HID])
        gg = jnp.tanh(g[:, 2 * HID:3 * HID])
        og = jax.nn.sigmoid(g[:, 3 * HID:])
        c = fg * c + ig * gg
        h = og * jnp.tanh(c)
    hr = h.reshape(_NB, 2 * HID)
    hm = (hr[:, :HID] + hr[:, HID:]) * 0.5
    av = a[0, 0]
    hp = jnp.where(hm > 0, hm, av * hm)
    xw = jnp.dot(hp, cw[:], preferred_element_type=F32)
    dinv = lax.rsqrt(c0[:, :1] + c1[:, :1] + 1.0)
    out[:] = xw * dinv


_lstm_call = pl.pallas_call(
    _lstm_body,
    grid=(_GRID,),
    in_specs=[pl.BlockSpec((_PB, NFEAT), functools.partial(lambda i, t: (t * _GRID + i, 0), t=t))
              for t in range(T)]
    + [
        pl.BlockSpec((NFEAT, 4 * HID), lambda i: (0, 0)),
        pl.BlockSpec((HID, 4 * HID), lambda i: (0, 0)),
        pl.BlockSpec((1, 4 * HID), lambda i: (0, 0)),
        pl.BlockSpec((HID, CONV), lambda i: (0, 0)),
        pl.BlockSpec((_NB, 16), lambda i: (i, 0)),
        pl.BlockSpec((_NB, 16), lambda i: (i, 0)),
        pl.BlockSpec(memory_space=pltpu.SMEM),
    ],
    out_specs=pl.BlockSpec((_NB, CONV), lambda i: (i, 0)),
    out_shape=jax.ShapeDtypeStruct((NP, CONV), F32),
)


def _fin_body(s0, s1, xw, c0, c1, b, out):
    acc = s0[:] + s1[:] + xw[:]
    dinv = lax.rsqrt(c0[:, :1] + c1[:, :1] + 1.0)
    v = acc * dinv + b[:]
    n = jnp.sqrt(jnp.sum(v * v, axis=1, keepdims=True))
    out[:] = v / jnp.maximum(n, 1e-12)


_fin_call = pl.pallas_call(
    _fin_body,
    grid=(4,),
    in_specs=[
        pl.BlockSpec((1024, CONV), lambda i: (i, 0)),
        pl.BlockSpec((1024, CONV), lambda i: (i, 0)),
        pl.BlockSpec((1024, CONV), lambda i: (i, 0)),
        pl.BlockSpec((1024, 16), lambda i: (i, 0)),
        pl.BlockSpec((1024, 16), lambda i: (i, 0)),
        pl.BlockSpec((1, CONV), lambda i: (0, 0)),
    ],
    out_specs=pl.BlockSpec((1024, CONV), lambda i: (i, 0)),
    out_shape=jax.ShapeDtypeStruct((NP, CONV), F32),
)


_LB = 512


def _score_body(ga, gb, w1a, w1b, b1, w2, b2, a, out):
    BF = jnp.bfloat16
    h = (jnp.dot(ga[:].astype(BF), w1a[:].astype(BF), preferred_element_type=F32)
         + jnp.dot(gb[:].astype(BF), w1b[:].astype(BF), preferred_element_type=F32) + b1[:])
    av = a[0, 0]
    h = jnp.where(h > 0, h, av * h)
    out[:] = jnp.dot(h, w2[:], preferred_element_type=F32) + b2[0, 0]


_score_call = pl.pallas_call(
    _score_body,
    grid=(LP // _LB,),
    in_specs=[
        pl.BlockSpec((_LB, 256), lambda i: (i, 0)),
        pl.BlockSpec((_LB, 256), lambda i: (i + LP // _LB, 0)),
        pl.BlockSpec((256, 256), lambda i: (0, 0)),
        pl.BlockSpec((256, 256), lambda i: (0, 0)),
        pl.BlockSpec((1, 256), lambda i: (0, 0)),
        pl.BlockSpec((256, 1), lambda i: (0, 0)),
        pl.BlockSpec(memory_space=pltpu.SMEM),
        pl.BlockSpec(memory_space=pltpu.SMEM),
    ],
    out_specs=pl.BlockSpec((_LB, 1), lambda i: (i, 0)),
    out_shape=jax.ShapeDtypeStruct((LP, 1), F32),
)


def kernel(x, edge_index, edge_attr, all_node_features, rel_features, paths, links,
           x_lin2_W, x_lin2_b, nn_cd_W, nn_cd_b, lstm_Wih, lstm_Whh, lstm_b,
           conv_W, conv_b, lin1_W, lin1_b, lin2_W, lin2_b, prelu_a):
    x_all = jnp.concatenate([all_node_features, rel_features], axis=0)
    src = edge_index[0]
    dst = edge_index[1]
    srcp = jnp.concatenate([src, jnp.zeros((EP - E,), I32)]).reshape(NW, -1, 128)
    dstp = jnp.concatenate([dst, jnp.full((EP - E,), DUMMY, I32)]).reshape(NW, -1, 128)
    pp = jnp.concatenate([paths, jnp.zeros((PP - P, T), I32)], axis=0)
    pidx = pp.T.reshape(NW, -1, 128)
    zeros64 = jnp.zeros((SH, CONV), F32)
    zeros16 = jnp.zeros((SH, 16), F32)
    ones16 = jnp.zeros((128, 16), F32).at[:, 0].set(1.0)
    a2 = prelu_a.reshape(1, 1)

    cnt = _deg_hist(dstp, zeros16, ones16)
    c0 = cnt[0, :NP]
    c1 = cnt[1, :NP]

    states = []
    for i in range(4):
        feats = _gather_paths(x_all, pidx)
        xwp = _lstm_call(*[feats] * T, lstm_Wih[i], lstm_Whh[i],
                         lstm_b[i].reshape(1, -1), conv_W[i], c0, c1, a2)
        S = _edge_scatter(xwp, srcp, dstp, zeros64)
        xc4 = _fin_call(S[0, :NP], S[1, :NP], xwp, c0, c1, conv_b[i].reshape(1, -1))
        states.append(xc4)
        x_pad = jnp.pad(xc4[:N_NODES], ((0, 0), (0, NFEAT - CONV)))
        x_all = x_all.at[2 * N_NODES:3 * N_NODES].set(x_pad)

    cs4 = jnp.concatenate(states, axis=1)
    cs = cs4[:N_NODES]

    l0 = jnp.pad(links[0], (0, LP - L))
    l1 = jnp.pad(links[1], (0, LP - L)) + 2000
    lidx = jnp.concatenate([l0, l1]).reshape(NW, -1, 128)
    g = _gather_links(cs4, lidx)
    outp = _score_call(g, g, lin1_W[:256], lin1_W[256:], lin1_b.reshape(1, -1),
                       lin2_W, lin2_b.reshape(1, 1), a2)
    out = outp[:L, 0]
    return (out, cs, x_all)


# speedup vs baseline: 1.0049x; 1.0049x over previous
import functools

import jax
import jax.numpy as jnp
from jax import lax
from jax.experimental import pallas as pl
from jax.experimental.pallas import tpu as pltpu
from jax.experimental.pallas import tpu_sc as plsc

F32 = jnp.float32
I32 = jnp.int32

N_NODES = 4000
NP = 4096
SH = 4224
DUMMY = 4096
NFEAT = 128
HID = 128
CONV = 64
T = 7
P = 8000
PP = 8192
FLAT = PP * T
E = 128000
EP = 131072
L = 10000
LP = 10240
NW = 32

_mesh = lambda: plsc.VectorSubcoreMesh(core_axis_name="c", subcore_axis_name="s")


@functools.cache
def _mk_gather(D, TOT, CH, ring, label):
    rw = TOT // NW // CH
    depth = max(ring // 2, 1)

    @functools.partial(
        pl.kernel,
        out_type=jax.ShapeDtypeStruct((TOT, D), F32),
        mesh=_mesh(),
        scratch_types=[
            pltpu.VMEM((rw, CH), I32),
            pltpu.VMEM((ring * CH, D), F32),
            pltpu.SemaphoreType.DMA,
            pltpu.SemaphoreType.DMA,
        ],
        compiler_params=pltpu.CompilerParams(use_tc_tiling_on_sc=False),
        name=label,
    )
    def k(table, idx, out, idx_v, rows_v, gsem, wsem):
        c = lax.axis_index("c")
        s = lax.axis_index("s")
        wid = s * 2 + c
        base = wid * rw
        pltpu.sync_copy(idx.at[wid], idx_v)

        def buf(j):
            return rows_v.at[pl.ds((j % ring) * CH, CH)]

        gd = {}
        wd = {}
        for j in range(min(depth, rw)):
            gd[j] = pltpu.async_copy(table.at[idx_v.at[j]], buf(j), gsem)
        for j in range(rw):
            gd[j].wait()
            wd[j] = pltpu.async_copy(buf(j), out.at[pl.ds((base + j) * CH, CH)], wsem)
            if j >= depth:
                wd[j - depth].wait()
            if j + depth < rw:
                gd[j + depth] = pltpu.async_copy(
                    table.at[idx_v.at[j + depth]], buf(j + depth), gsem)
        for j in range(max(rw - depth, 0), rw):
            wd[j].wait()

    return k


def _gather_paths(table, idx):
    return _mk_gather(NFEAT, FLAT, 256, 2, "path_gather")(table, idx)


def _gather_links(table, idx):
    return _mk_gather(256, 2 * LP, 128, 2, "link_gather")(table, idx)


_SROWS = SH // 16


@functools.cache
def _mk_edge_scatter():
    @functools.partial(
        pl.kernel,
        out_type=jax.ShapeDtypeStruct((2, SH, CONV), F32),
        mesh=_mesh(),
        scratch_types=[
            pltpu.VMEM((EP // 256 // NW, 256), I32),
            pltpu.VMEM((EP // 256 // NW, 256), I32),
            pltpu.VMEM((4 * 256, CONV), F32),
            pltpu.VMEM_SHARED((SH, CONV), F32),
            pltpu.SemaphoreType.DMA,
            pltpu.SemaphoreType.DMA,
        ],
        compiler_params=pltpu.CompilerParams(use_tc_tiling_on_sc=False),
        name="edge_scatter",
    )
    def k(xw, sI, dI, z, out, sv, dv, rows, shared, gsem, ssem):
        c = lax.axis_index("c")
        s = lax.axis_index("s")
        wid = s * 2 + c
        nchunk = EP // 256 // NW
        ring, depth = 4, 2
        pltpu.sync_copy(z.at[pl.ds(s * _SROWS, _SROWS)], shared.at[pl.ds(s * _SROWS, _SROWS)])
        pltpu.sync_copy(sI.at[wid], sv)
        pltpu.sync_copy(dI.at[wid], dv)
        plsc.subcore_barrier()

        def buf(j):
            return rows.at[pl.ds((j % ring) * 256, 256)]

        gd = {}
        sd = {}
        for j in range(depth):
            gd[j] = pltpu.async_copy(xw.at[sv.at[j]], buf(j), gsem)
        for j in range(nchunk):
            gd[j].wait()
            sd[j] = pltpu.async_copy(buf(j), shared.at[dv.at[j]], ssem, add=True)
            if j >= depth:
                sd[j - depth].wait()
            if j + depth < nchunk:
                gd[j + depth] = pltpu.async_copy(
                    xw.at[sv.at[j + depth]], buf(j + depth), gsem)
        for j in range(nchunk - depth, nchunk):
            sd[j].wait()
        plsc.subcore_barrier()
        pltpu.sync_copy(shared.at[pl.ds(s * _SROWS, _SROWS)], out.at[c, pl.ds(s * _SROWS, _SROWS)])

    return k


def _edge_scatter(xw, sI, dI, z):
    return _mk_edge_scatter()(xw, sI, dI, z)


@functools.cache
def _mk_deg_hist():
    @functools.partial(
        pl.kernel,
        out_type=jax.ShapeDtypeStruct((2, SH, 16), F32),
        mesh=_mesh(),
        scratch_types=[
            pltpu.VMEM((EP // 256 // NW, 256), I32),
            pltpu.VMEM((256, 16), F32),
            pltpu.VMEM_SHARED((SH, 16), F32),
            pltpu.SemaphoreType.DMA,
        ],
        compiler_params=pltpu.CompilerParams(use_tc_tiling_on_sc=False),
        name="deg_hist",
    )
    def k(dI, z, ones, out, dv, ones_v, shared, sem):
        c = lax.axis_index("c")
        s = lax.axis_index("s")
        wid = s * 2 + c
        nchunk = EP // 256 // NW
        pltpu.sync_copy(z.at[pl.ds(s * _SROWS, _SROWS)], shared.at[pl.ds(s * _SROWS, _SROWS)])
        pltpu.sync_copy(dI.at[wid], dv)
        pltpu.sync_copy(ones, ones_v)
        plsc.subcore_barrier()

        sd = {}
        for j in range(nchunk):
            sd[j] = pltpu.async_copy(ones_v, shared.at[dv.at[j]], sem, add=True)
            if j >= 8:
                sd[j - 8].wait()
        for j in range(nchunk - 8, nchunk):
            sd[j].wait()
        plsc.subcore_barrier()
        pltpu.sync_copy(shared.at[pl.ds(s * _SROWS, _SROWS)], out.at[c, pl.ds(s * _SROWS, _SROWS)])

    return k


def _deg_hist(dI, z, ones):
    return _mk_deg_hist()(dI, z, ones)


_PB = 512
_NB = 256
_GRID = PP // _PB


def _lstm_body(f0, f1, f2, f3, f4, f5, f6, wih, whh, b, cw, c0, c1, a, out):
    fs = (f0, f1, f2, f3, f4, f5, f6)
    BF = jnp.bfloat16
    W_ih = wih[:].astype(BF)
    W_hh = whh[:].astype(BF)
    bb = b[:]
    h = jnp.zeros((_PB, HID), F32)
    c = jnp.zeros((_PB, HID), F32)
    for t in range(T):
        xt = fs[t][:].astype(BF)
        g = (jnp.dot(xt, W_ih, preferred_element_type=F32)
             + jnp.dot(h.astype(BF), W_hh, preferred_element_type=F32) + bb)
        ig = jax.nn.sigmoid(g[:, :HID])
        fg = jax.nn.sigmoid(g[:, HID:2 * HID])
        gg = jnp.tanh(g[:, 2 * HID:3 * HID])
        og = jax.nn.sigmoid(g[:, 3 * HID:])
        c = fg * c + ig * gg
        h = og * jnp.tanh(c)
    hr = h.reshape(_NB, 2 * HID)
    hm = (hr[:, :HID] + hr[:, HID:]) * 0.5
    av = a[0, 0]
    hp = jnp.where(hm > 0, hm, av * hm)
    xw = jnp.dot(hp, cw[:], preferred_element_type=F32)
    dinv = lax.rsqrt(c0[:, :1] + c1[:, :1] + 1.0)
    out[:] = xw * dinv


_lstm_call = pl.pallas_call(
    _lstm_body,
    grid=(_GRID,),
    in_specs=[pl.BlockSpec((_PB, NFEAT), functools.partial(lambda i, t: (t * _GRID + i, 0), t=t))
              for t in range(T)]
    + [
        pl.BlockSpec((NFEAT, 4 * HID), lambda i: (0, 0)),
        pl.BlockSpec((HID, 4 * HID), lambda i: (0, 0)),
        pl.BlockSpec((1, 4 * HID), lambda i: (0, 0)),
        pl.BlockSpec((HID, CONV), lambda i: (0, 0)),
        pl.BlockSpec((_NB, 16), lambda i: (i, 0)),
        pl.BlockSpec((_NB, 16), lambda i: (i, 0)),
        pl.BlockSpec(memory_space=pltpu.SMEM),
    ],
    out_specs=pl.BlockSpec((_NB, CONV), lambda i: (i, 0)),
    out_shape=jax.ShapeDtypeStruct((NP, CONV), F32),
)


def _fin_body(s0, s1, xw, c0, c1, b, out):
    acc = s0[:] + s1[:] + xw[:]
    dinv = lax.rsqrt(c0[:, :1] + c1[:, :1] + 1.0)
    v = acc * dinv + b[:]
    n = jnp.sqrt(jnp.sum(v * v, axis=1, keepdims=True))
    out[:] = v / jnp.maximum(n, 1e-12)


_fin_call = pl.pallas_call(
    _fin_body,
    grid=(4,),
    in_specs=[
        pl.BlockSpec((1024, CONV), lambda i: (i, 0)),
        pl.BlockSpec((1024, CONV), lambda i: (i, 0)),
        pl.BlockSpec((1024, CONV), lambda i: (i, 0)),
        pl.BlockSpec((1024, 16), lambda i: (i, 0)),
        pl.BlockSpec((1024, 16), lambda i: (i, 0)),
        pl.BlockSpec((1, CONV), lambda i: (0, 0)),
    ],
    out_specs=pl.BlockSpec((1024, CONV), lambda i: (i, 0)),
    out_shape=jax.ShapeDtypeStruct((NP, CONV), F32),
)


_LB = 512


def _score_body(ga, gb, w1a, w1b, b1, w2, b2, a, out):
    BF = jnp.bfloat16
    h = (jnp.dot(ga[:].astype(BF), w1a[:].astype(BF), preferred_element_type=F32)
         + jnp.dot(gb[:].astype(BF), w1b[:].astype(BF), preferred_element_type=F32) + b1[:])
    av = a[0, 0]
    h = jnp.where(h > 0, h, av * h)
    out[:] = jnp.dot(h, w2[:], preferred_element_type=F32) + b2[0, 0]


_score_call = pl.pallas_call(
    _score_body,
    grid=(LP // _LB,),
    in_specs=[
        pl.BlockSpec((_LB, 256), lambda i: (i, 0)),
        pl.BlockSpec((_LB, 256), lambda i: (i + LP // _LB, 0)),
        pl.BlockSpec((256, 256), lambda i: (0, 0)),
        pl.BlockSpec((256, 256), lambda i: (0, 0)),
        pl.BlockSpec((1, 256), lambda i: (0, 0)),
        pl.BlockSpec((256, 1), lambda i: (0, 0)),
        pl.BlockSpec(memory_space=pltpu.SMEM),
        pl.BlockSpec(memory_space=pltpu.SMEM),
    ],
    out_specs=pl.BlockSpec((_LB, 1), lambda i: (i, 0)),
    out_shape=jax.ShapeDtypeStruct((LP, 1), F32),
)


def kernel(x, edge_index, edge_attr, all_node_features, rel_features, paths, links,
           x_lin2_W, x_lin2_b, nn_cd_W, nn_cd_b, lstm_Wih, lstm_Whh, lstm_b,
           conv_W, conv_b, lin1_W, lin1_b, lin2_W, lin2_b, prelu_a):
    x_all = jnp.concatenate([all_node_features, rel_features], axis=0)
    src = edge_index[0]
    dst = edge_index[1]
    srcp = jnp.concatenate([src, jnp.zeros((EP - E,), I32)]).reshape(NW, -1, 256)
    dstp = jnp.concatenate([dst, jnp.full((EP - E,), DUMMY, I32)]).reshape(NW, -1, 256)
    pp = jnp.concatenate([paths, jnp.zeros((PP - P, T), I32)], axis=0)
    pidx = pp.T.reshape(NW, -1, 256)
    zeros64 = jnp.zeros((SH, CONV), F32)
    zeros16 = jnp.zeros((SH, 16), F32)
    ones16 = jnp.zeros((256, 16), F32).at[:, 0].set(1.0)
    a2 = prelu_a.reshape(1, 1)

    cnt = _deg_hist(dstp, zeros16, ones16)
    c0 = cnt[0, :NP]
    c1 = cnt[1, :NP]

    states = []
    for i in range(4):
        feats = _gather_paths(x_all, pidx)
        xwp = _lstm_call(*[feats] * T, lstm_Wih[i], lstm_Whh[i],
                         lstm_b[i].reshape(1, -1), conv_W[i], c0, c1, a2)
        S = _edge_scatter(xwp, srcp, dstp, zeros64)
        xc4 = _fin_call(S[0, :NP], S[1, :NP], xwp, c0, c1, conv_b[i].reshape(1, -1))
        states.append(xc4)
        x_pad = jnp.pad(xc4[:N_NODES], ((0, 0), (0, NFEAT - CONV)))
        x_all = x_all.at[2 * N_NODES:3 * N_NODES].set(x_pad)

    cs4 = jnp.concatenate(states, axis=1)
    cs = cs4[:N_NODES]

    l0 = jnp.pad(links[0], (0, LP - L))
    l1 = jnp.pad(links[1], (0, LP - L)) + 2000
    lidx = jnp.concatenate([l0, l1]).reshape(NW, -1, 128)
    g = _gather_links(cs4, lidx)
    outp = _score_call(g, g, lin1_W[:256], lin1_W[256:], lin1_b.reshape(1, -1),
                       lin2_W, lin2_b.reshape(1, 1), a2)
    out = outp[:L, 0]
    return (out, cs, x_all)


# speedup vs baseline: 1.0066x; 1.0017x over previous
import functools

import jax
import jax.numpy as jnp
from jax import lax
from jax.experimental import pallas as pl
from jax.experimental.pallas import tpu as pltpu
from jax.experimental.pallas import tpu_sc as plsc

F32 = jnp.float32
I32 = jnp.int32

N_NODES = 4000
NP = 4096
SH = 4224
DUMMY = 4096
NFEAT = 128
HID = 128
CONV = 64
T = 7
P = 8000
PP = 8192
FLAT = PP * T
E = 128000
EP = 131072
L = 10000
LP = 10240
NW = 32

_mesh = lambda: plsc.VectorSubcoreMesh(core_axis_name="c", subcore_axis_name="s")


@functools.cache
def _mk_gather(D, TOT, CH, ring, label):
    rw = TOT // NW // CH
    depth = max(ring // 2, 1)

    @functools.partial(
        pl.kernel,
        out_type=jax.ShapeDtypeStruct((TOT, D), F32),
        mesh=_mesh(),
        scratch_types=[
            pltpu.VMEM((rw, CH), I32),
            pltpu.VMEM((ring * CH, D), F32),
            pltpu.SemaphoreType.DMA,
            pltpu.SemaphoreType.DMA,
        ],
        compiler_params=pltpu.CompilerParams(use_tc_tiling_on_sc=False),
        name=label,
    )
    def k(table, idx, out, idx_v, rows_v, gsem, wsem):
        c = lax.axis_index("c")
        s = lax.axis_index("s")
        wid = s * 2 + c
        base = wid * rw
        pltpu.sync_copy(idx.at[wid], idx_v)

        def buf(j):
            return rows_v.at[pl.ds((j % ring) * CH, CH)]

        gd = {}
        wd = {}
        for j in range(min(depth, rw)):
            gd[j] = pltpu.async_copy(table.at[idx_v.at[j]], buf(j), gsem)
        for j in range(rw):
            gd[j].wait()
            wd[j] = pltpu.async_copy(buf(j), out.at[pl.ds((base + j) * CH, CH)], wsem)
            if j >= depth:
                wd[j - depth].wait()
            if j + depth < rw:
                gd[j + depth] = pltpu.async_copy(
                    table.at[idx_v.at[j + depth]], buf(j + depth), gsem)
        for j in range(max(rw - depth, 0), rw):
            wd[j].wait()

    return k


def _gather_paths(table, idx):
    return _mk_gather(NFEAT, FLAT, 256, 2, "path_gather")(table, idx)


def _gather_links(table, idx):
    return _mk_gather(256, 2 * LP, 128, 2, "link_gather")(table, idx)


_SROWS = SH // 16


@functools.cache
def _mk_edge_scatter():
    @functools.partial(
        pl.kernel,
        out_type=jax.ShapeDtypeStruct((2, SH, CONV), F32),
        mesh=_mesh(),
        scratch_types=[
            pltpu.VMEM((EP // 256 // NW, 256), I32),
            pltpu.VMEM((EP // 256 // NW, 256), I32),
            pltpu.VMEM((4 * 256, CONV), F32),
            pltpu.VMEM_SHARED((SH, CONV), F32),
            pltpu.SemaphoreType.DMA,
            pltpu.SemaphoreType.DMA,
        ],
        compiler_params=pltpu.CompilerParams(use_tc_tiling_on_sc=False),
        name="edge_scatter",
    )
    def k(xw, sI, dI, z, out, sv, dv, rows, shared, gsem, ssem):
        c = lax.axis_index("c")
        s = lax.axis_index("s")
        wid = s * 2 + c
        nchunk = EP // 256 // NW
        ring, depth = 4, 2
        pltpu.sync_copy(z.at[pl.ds(s * _SROWS, _SROWS)], shared.at[pl.ds(s * _SROWS, _SROWS)])
        pltpu.sync_copy(sI.at[wid], sv)
        pltpu.sync_copy(dI.at[wid], dv)
        plsc.subcore_barrier()

        def buf(j):
            return rows.at[pl.ds((j % ring) * 256, 256)]

        gd = {}
        sd = {}
        for j in range(depth):
            gd[j] = pltpu.async_copy(xw.at[sv.at[j]], buf(j), gsem)
        for j in range(nchunk):
            gd[j].wait()
            sd[j] = pltpu.async_copy(buf(j), shared.at[dv.at[j]], ssem, add=True)
            if j >= depth:
                sd[j - depth].wait()
            if j + depth < nchunk:
                gd[j + depth] = pltpu.async_copy(
                    xw.at[sv.at[j + depth]], buf(j + depth), gsem)
        for j in range(nchunk - depth, nchunk):
            sd[j].wait()
        plsc.subcore_barrier()
        pltpu.sync_copy(shared.at[pl.ds(s * _SROWS, _SROWS)], out.at[c, pl.ds(s * _SROWS, _SROWS)])

    return k


def _edge_scatter(xw, sI, dI, z):
    return _mk_edge_scatter()(xw, sI, dI, z)


@functools.cache
def _mk_deg_hist():
    @functools.partial(
        pl.kernel,
        out_type=jax.ShapeDtypeStruct((2, SH, 16), F32),
        mesh=_mesh(),
        scratch_types=[
            pltpu.VMEM((EP // 256 // NW, 256), I32),
            pltpu.VMEM((256, 16), F32),
            pltpu.VMEM_SHARED((SH, 16), F32),
            pltpu.SemaphoreType.DMA,
        ],
        compiler_params=pltpu.CompilerParams(use_tc_tiling_on_sc=False),
        name="deg_hist",
    )
    def k(dI, z, ones, out, dv, ones_v, shared, sem):
        c = lax.axis_index("c")
        s = lax.axis_index("s")
        wid = s * 2 + c
        nchunk = EP // 256 // NW
        pltpu.sync_copy(z.at[pl.ds(s * _SROWS, _SROWS)], shared.at[pl.ds(s * _SROWS, _SROWS)])
        pltpu.sync_copy(dI.at[wid], dv)
        pltpu.sync_copy(ones, ones_v)
        plsc.subcore_barrier()

        sd = {}
        for j in range(nchunk):
            sd[j] = pltpu.async_copy(ones_v, shared.at[dv.at[j]], sem, add=True)
            if j >= 8:
                sd[j - 8].wait()
        for j in range(nchunk - 8, nchunk):
            sd[j].wait()
        plsc.subcore_barrier()
        pltpu.sync_copy(shared.at[pl.ds(s * _SROWS, _SROWS)], out.at[c, pl.ds(s * _SROWS, _SROWS)])

    return k


def _deg_hist(dI, z, ones):
    return _mk_deg_hist()(dI, z, ones)


_PB = 512
_NB = 256
_GRID = PP // _PB


def _lstm_body(f0, f1, f2, f3, f4, f5, f6, wih, whh, b, cw, c0, c1, a, out):
    fs = (f0, f1, f2, f3, f4, f5, f6)
    BF = jnp.bfloat16
    W_ih = wih[:].astype(BF)
    W_hh = whh[:].astype(BF)
    bb = b[:]
    h = jnp.zeros((_PB, HID), F32)
    c = jnp.zeros((_PB, HID), F32)
    for t in range(T):
        xt = fs[t][:].astype(BF)
        g = (jnp.dot(xt, W_ih, preferred_element_type=F32)
             + jnp.dot(h.astype(BF), W_hh, preferred_element_type=F32) + bb)
        ig = jax.nn.sigmoid(g[:, :HID])
        fg = jax.nn.sigmoid(g[:, HID:2 * HID])
        gg = jnp.tanh(g[:, 2 * HID:3 * HID])
        og = jax.nn.sigmoid(g[:, 3 * HID:])
        c = fg * c + ig * gg
        h = og * jnp.tanh(c)
    hr = h.reshape(_NB, 2 * HID)
    hm = (hr[:, :HID] + hr[:, HID:]) * 0.5
    av = a[0, 0]
    hp = jnp.where(hm > 0, hm, av * hm)
    xw = jnp.dot(hp, cw[:], preferred_element_type=F32)
    dinv = lax.rsqrt(c0[:, :1] + c1[:, :1] + 1.0)
    out[:] = xw * dinv


_lstm_call = pl.pallas_call(
    _lstm_body,
    grid=(_GRID,),
    in_specs=[pl.BlockSpec((_PB, NFEAT), functools.partial(lambda i, t: (t * _GRID + i, 0), t=t))
              for t in range(T)]
    + [
        pl.BlockSpec((NFEAT, 4 * HID), lambda i: (0, 0)),
        pl.BlockSpec((HID, 4 * HID), lambda i: (0, 0)),
        pl.BlockSpec((1, 4 * HID), lambda i: (0, 0)),
        pl.BlockSpec((HID, CONV), lambda i: (0, 0)),
        pl.BlockSpec((_NB, 16), lambda i: (i, 0)),
        pl.BlockSpec((_NB, 16), lambda i: (i, 0)),
        pl.BlockSpec(memory_space=pltpu.SMEM),
    ],
    out_specs=pl.BlockSpec((_NB, CONV), lambda i: (i, 0)),
    out_shape=jax.ShapeDtypeStruct((NP, CONV), F32),
)


def _fin_body(s0, s1, xw, c0, c1, b, out):
    acc = s0[:] + s1[:] + xw[:]
    dinv = lax.rsqrt(c0[:, :1] + c1[:, :1] + 1.0)
    v = acc * dinv + b[:]
    n = jnp.sqrt(jnp.sum(v * v, axis=1, keepdims=True))
    out[:] = v / jnp.maximum(n, 1e-12)


_fin_call = pl.pallas_call(
    _fin_body,
    grid=(4,),
    in_specs=[
        pl.BlockSpec((1024, CONV), lambda i: (i, 0)),
        pl.BlockSpec((1024, CONV), lambda i: (i, 0)),
        pl.BlockSpec((1024, CONV), lambda i: (i, 0)),
        pl.BlockSpec((1024, 16), lambda i: (i, 0)),
        pl.BlockSpec((1024, 16), lambda i: (i, 0)),
        pl.BlockSpec((1, CONV), lambda i: (0, 0)),
    ],
    out_specs=pl.BlockSpec((1024, CONV), lambda i: (i, 0)),
    out_shape=jax.ShapeDtypeStruct((NP, CONV), F32),
)


_LB = 512


def _score_body(ga, gb, w1a, w1b, b1, w2, b2, a, out):
    BF = jnp.bfloat16
    h = (jnp.dot(ga[:].astype(BF), w1a[:].astype(BF), preferred_element_type=F32)
         + jnp.dot(gb[:].astype(BF), w1b[:].astype(BF), preferred_element_type=F32) + b1[:])
    av = a[0, 0]
    h = jnp.where(h > 0, h, av * h)
    out[:] = jnp.dot(h, w2[:], preferred_element_type=F32) + b2[0, 0]


_score_call = pl.pallas_call(
    _score_body,
    grid=(LP // _LB,),
    in_specs=[
        pl.BlockSpec((_LB, 256), lambda i: (i, 0)),
        pl.BlockSpec((_LB, 256), lambda i: (i + LP // _LB, 0)),
        pl.BlockSpec((256, 256), lambda i: (0, 0)),
        pl.BlockSpec((256, 256), lambda i: (0, 0)),
        pl.BlockSpec((1, 256), lambda i: (0, 0)),
        pl.BlockSpec((256, 1), lambda i: (0, 0)),
        pl.BlockSpec(memory_space=pltpu.SMEM),
        pl.BlockSpec(memory_space=pltpu.SMEM),
    ],
    out_specs=pl.BlockSpec((_LB, 1), lambda i: (i, 0)),
    out_shape=jax.ShapeDtypeStruct((LP, 1), F32),
)


def kernel(x, edge_index, edge_attr, all_node_features, rel_features, paths, links,
           x_lin2_W, x_lin2_b, nn_cd_W, nn_cd_b, lstm_Wih, lstm_Whh, lstm_b,
           conv_W, conv_b, lin1_W, lin1_b, lin2_W, lin2_b, prelu_a):
    x_all = jnp.concatenate([all_node_features, rel_features], axis=0)
    src = edge_index[0]
    dst = edge_index[1]
    srcp = jnp.concatenate([src, jnp.zeros((EP - E,), I32)]).reshape(NW, -1, 256)
    dum = DUMMY + (jnp.arange(EP - E, dtype=I32) % (SH - NP))
    dstp = jnp.concatenate([dst, dum]).reshape(NW, -1, 256)
    pp = jnp.concatenate([paths, jnp.zeros((PP - P, T), I32)], axis=0)
    pidx = pp.T.reshape(NW, -1, 256)
    zeros64 = jnp.zeros((SH, CONV), F32)
    zeros16 = jnp.zeros((SH, 16), F32)
    ones16 = jnp.zeros((256, 16), F32).at[:, 0].set(1.0)
    a2 = prelu_a.reshape(1, 1)

    cnt = _deg_hist(dstp, zeros16, ones16)
    c0 = cnt[0, :NP]
    c1 = cnt[1, :NP]

    states = []
    for i in range(4):
        feats = _gather_paths(x_all, pidx)
        xwp = _lstm_call(*[feats] * T, lstm_Wih[i], lstm_Whh[i],
                         lstm_b[i].reshape(1, -1), conv_W[i], c0, c1, a2)
        S = _edge_scatter(xwp, srcp, dstp, zeros64)
        xc4 = _fin_call(S[0, :NP], S[1, :NP], xwp, c0, c1, conv_b[i].reshape(1, -1))
        states.append(xc4)
        x_pad = jnp.pad(xc4[:N_NODES], ((0, 0), (0, NFEAT - CONV)))
        x_all = x_all.at[2 * N_NODES:3 * N_NODES].set(x_pad)

    cs4 = jnp.concatenate(states, axis=1)
    cs = cs4[:N_NODES]

    l0 = jnp.pad(links[0], (0, LP - L))
    l1 = jnp.pad(links[1], (0, LP - L)) + 2000
    lidx = jnp.concatenate([l0, l1]).reshape(NW, -1, 128)
    g = _gather_links(cs4, lidx)
    outp = _score_call(g, g, lin1_W[:256], lin1_W[256:], lin1_b.reshape(1, -1),
                       lin2_W, lin2_b.reshape(1, 1), a2)
    out = outp[:L, 0]
    return (out, cs, x_all)


# speedup vs baseline: 1.3164x; 1.3077x over previous
import functools

import jax
import jax.numpy as jnp
from jax import lax
from jax.experimental import pallas as pl
from jax.experimental.pallas import tpu as pltpu
from jax.experimental.pallas import tpu_sc as plsc

F32 = jnp.float32
I32 = jnp.int32

N_NODES = 4000
NP = 4096
SH = 4224
DUMMY = 4096
NFEAT = 128
HID = 128
CONV = 64
T = 7
P = 8000
PP = 8192
FLAT = PP * T
E = 128000
EP = 131072
L = 10000
LP = 10240
NW = 32

_mesh = lambda: plsc.VectorSubcoreMesh(core_axis_name="c", subcore_axis_name="s")


@functools.cache
def _mk_gather(D, TOT, CH, ring, V, label, stage=True):
    rw = TOT // NW // CH
    depth = max(ring // 2, 1)
    rps = V // 16

    @functools.partial(
        pl.kernel,
        out_type=jax.ShapeDtypeStruct((TOT, D), F32),
        mesh=_mesh(),
        scratch_types=[pltpu.VMEM((rw, CH), I32), pltpu.VMEM((ring * CH, D), F32)]
        + ([pltpu.VMEM_SHARED((V, D), F32)] if stage else [])
        + [pltpu.SemaphoreType.DMA, pltpu.SemaphoreType.DMA],
        compiler_params=pltpu.CompilerParams(use_tc_tiling_on_sc=False),
        name=label,
    )
    def k(table, idx, out, *refs):
        idx_v, rows_v = refs[0], refs[1]
        gsem, wsem = refs[-2], refs[-1]
        c = lax.axis_index("c")
        s = lax.axis_index("s")
        wid = s * 2 + c
        base = wid * rw
        pltpu.sync_copy(idx.at[wid], idx_v)
        if stage:
            tab = refs[2]
            pltpu.sync_copy(table.at[pl.ds(s * rps, rps)], tab.at[pl.ds(s * rps, rps)])
            plsc.subcore_barrier()
        else:
            tab = table

        def buf(j):
            return rows_v.at[pl.ds((j % ring) * CH, CH)]

        gd = {}
        wd = {}
        for j in range(min(depth, rw)):
            gd[j] = pltpu.async_copy(tab.at[idx_v.at[j]], buf(j), gsem)
        for j in range(rw):
            gd[j].wait()
            wd[j] = pltpu.async_copy(buf(j), out.at[pl.ds((base + j) * CH, CH)], wsem)
            if j >= depth:
                wd[j - depth].wait()
            if j + depth < rw:
                gd[j + depth] = pltpu.async_copy(
                    tab.at[idx_v.at[j + depth]], buf(j + depth), gsem)
        for j in range(max(rw - depth, 0), rw):
            wd[j].wait()

    return k


VTAB = 12032


def _gather_paths(table, idx):
    return _mk_gather(NFEAT, FLAT, 256, 2, VTAB, "path_gather", stage=False)(table, idx)


def _gather_links(table, idx):
    return _mk_gather(256, 2 * LP, 128, 2, NP, "link_gather", stage=False)(table, idx)


_SROWS = SH // 16


@functools.cache
def _mk_edge_scatter():
    @functools.partial(
        pl.kernel,
        out_type=jax.ShapeDtypeStruct((2, SH, CONV), F32),
        mesh=_mesh(),
        scratch_types=[
            pltpu.VMEM((EP // 256 // NW, 256), I32),
            pltpu.VMEM((EP // 256 // NW, 256), I32),
            pltpu.VMEM((4 * 256, CONV), F32),
            pltpu.VMEM_SHARED((SH, CONV), F32),
            pltpu.VMEM_SHARED((NP, CONV), F32),
            pltpu.SemaphoreType.DMA,
            pltpu.SemaphoreType.DMA,
        ],
        compiler_params=pltpu.CompilerParams(use_tc_tiling_on_sc=False),
        name="edge_scatter",
    )
    def k(xw, sI, dI, z, out, sv, dv, rows, shared, tab, gsem, ssem):
        c = lax.axis_index("c")
        s = lax.axis_index("s")
        wid = s * 2 + c
        nchunk = EP // 256 // NW
        ring, depth = 4, 2
        pltpu.sync_copy(z.at[pl.ds(s * _SROWS, _SROWS)], shared.at[pl.ds(s * _SROWS, _SROWS)])
        pltpu.sync_copy(xw.at[pl.ds(s * (NP // 16), NP // 16)], tab.at[pl.ds(s * (NP // 16), NP // 16)])
        pltpu.sync_copy(sI.at[wid], sv)
        pltpu.sync_copy(dI.at[wid], dv)
        plsc.subcore_barrier()

        def buf(j):
            return rows.at[pl.ds((j % ring) * 256, 256)]

        gd = {}
        sd = {}
        for j in range(depth):
            gd[j] = pltpu.async_copy(tab.at[sv.at[j]], buf(j), gsem)
        for j in range(nchunk):
            gd[j].wait()
            sd[j] = pltpu.async_copy(buf(j), shared.at[dv.at[j]], ssem, add=True)
            if j >= depth:
                sd[j - depth].wait()
            if j + depth < nchunk:
                gd[j + depth] = pltpu.async_copy(
                    tab.at[sv.at[j + depth]], buf(j + depth), gsem)
        for j in range(nchunk - depth, nchunk):
            sd[j].wait()
        plsc.subcore_barrier()
        pltpu.sync_copy(shared.at[pl.ds(s * _SROWS, _SROWS)], out.at[c, pl.ds(s * _SROWS, _SROWS)])

    return k


def _edge_scatter(xw, sI, dI, z):
    return _mk_edge_scatter()(xw, sI, dI, z)


@functools.cache
def _mk_deg_hist():
    @functools.partial(
        pl.kernel,
        out_type=jax.ShapeDtypeStruct((2, SH, 16), F32),
        mesh=_mesh(),
        scratch_types=[
            pltpu.VMEM((EP // 256 // NW, 256), I32),
            pltpu.VMEM((256, 16), F32),
            pltpu.VMEM_SHARED((SH, 16), F32),
            pltpu.SemaphoreType.DMA,
        ],
        compiler_params=pltpu.CompilerParams(use_tc_tiling_on_sc=False),
        name="deg_hist",
    )
    def k(dI, z, ones, out, dv, ones_v, shared, sem):
        c = lax.axis_index("c")
        s = lax.axis_index("s")
        wid = s * 2 + c
        nchunk = EP // 256 // NW
        pltpu.sync_copy(z.at[pl.ds(s * _SROWS, _SROWS)], shared.at[pl.ds(s * _SROWS, _SROWS)])
        pltpu.sync_copy(dI.at[wid], dv)
        pltpu.sync_copy(ones, ones_v)
        plsc.subcore_barrier()

        sd = {}
        for j in range(nchunk):
            sd[j] = pltpu.async_copy(ones_v, shared.at[dv.at[j]], sem, add=True)
            if j >= 8:
                sd[j - 8].wait()
        for j in range(nchunk - 8, nchunk):
            sd[j].wait()
        plsc.subcore_barrier()
        pltpu.sync_copy(shared.at[pl.ds(s * _SROWS, _SROWS)], out.at[c, pl.ds(s * _SROWS, _SROWS)])

    return k


def _deg_hist(dI, z, ones):
    return _mk_deg_hist()(dI, z, ones)


_PB = 512
_NB = 256
_GRID = PP // _PB


def _lstm_body(f0, f1, f2, f3, f4, f5, f6, wih, whh, b, cw, c0, c1, a, out):
    fs = (f0, f1, f2, f3, f4, f5, f6)
    BF = jnp.bfloat16
    W_ih = wih[:].astype(BF)
    W_hh = whh[:].astype(BF)
    bb = b[:]
    h = jnp.zeros((_PB, HID), F32)
    c = jnp.zeros((_PB, HID), F32)
    for t in range(T):
        xt = fs[t][:].astype(BF)
        g = (jnp.dot(xt, W_ih, preferred_element_type=F32)
             + jnp.dot(h.astype(BF), W_hh, preferred_element_type=F32) + bb)
        ig = jax.nn.sigmoid(g[:, :HID])
        fg = jax.nn.sigmoid(g[:, HID:2 * HID])
        gg = jnp.tanh(g[:, 2 * HID:3 * HID])
        og = jax.nn.sigmoid(g[:, 3 * HID:])
        c = fg * c + ig * gg
        h = og * jnp.tanh(c)
    hr = h.reshape(_NB, 2 * HID)
    hm = (hr[:, :HID] + hr[:, HID:]) * 0.5
    av = a[0, 0]
    hp = jnp.where(hm > 0, hm, av * hm)
    xw = jnp.dot(hp, cw[:], preferred_element_type=F32)
    dinv = lax.rsqrt(c0[:, :1] + c1[:, :1] + 1.0)
    out[:] = xw * dinv


_lstm_call = pl.pallas_call(
    _lstm_body,
    grid=(_GRID,),
    in_specs=[pl.BlockSpec((_PB, NFEAT), functools.partial(lambda i, t: (t * _GRID + i, 0), t=t))
              for t in range(T)]
    + [
        pl.BlockSpec((NFEAT, 4 * HID), lambda i: (0, 0)),
        pl.BlockSpec((HID, 4 * HID), lambda i: (0, 0)),
        pl.BlockSpec((1, 4 * HID), lambda i: (0, 0)),
        pl.BlockSpec((HID, CONV), lambda i: (0, 0)),
        pl.BlockSpec((_NB, 16), lambda i: (i, 0)),
        pl.BlockSpec((_NB, 16), lambda i: (i, 0)),
        pl.BlockSpec(memory_space=pltpu.SMEM),
    ],
    out_specs=pl.BlockSpec((_NB, CONV), lambda i: (i, 0)),
    out_shape=jax.ShapeDtypeStruct((NP, CONV), F32),
)


def _fin_body(s0, s1, xw, c0, c1, b, out):
    acc = s0[:] + s1[:] + xw[:]
    dinv = lax.rsqrt(c0[:, :1] + c1[:, :1] + 1.0)
    v = acc * dinv + b[:]
    n = jnp.sqrt(jnp.sum(v * v, axis=1, keepdims=True))
    out[:] = v / jnp.maximum(n, 1e-12)


_fin_call = pl.pallas_call(
    _fin_body,
    grid=(4,),
    in_specs=[
        pl.BlockSpec((1024, CONV), lambda i: (i, 0)),
        pl.BlockSpec((1024, CONV), lambda i: (i, 0)),
        pl.BlockSpec((1024, CONV), lambda i: (i, 0)),
        pl.BlockSpec((1024, 16), lambda i: (i, 0)),
        pl.BlockSpec((1024, 16), lambda i: (i, 0)),
        pl.BlockSpec((1, CONV), lambda i: (0, 0)),
    ],
    out_specs=pl.BlockSpec((1024, CONV), lambda i: (i, 0)),
    out_shape=jax.ShapeDtypeStruct((NP, CONV), F32),
)


_LB = 512


def _score_body(ga, gb, w1a, w1b, b1, w2, b2, a, out):
    BF = jnp.bfloat16
    h = (jnp.dot(ga[:].astype(BF), w1a[:].astype(BF), preferred_element_type=F32)
         + jnp.dot(gb[:].astype(BF), w1b[:].astype(BF), preferred_element_type=F32) + b1[:])
    av = a[0, 0]
    h = jnp.where(h > 0, h, av * h)
    out[:] = jnp.dot(h, w2[:], preferred_element_type=F32) + b2[0, 0]


_score_call = pl.pallas_call(
    _score_body,
    grid=(LP // _LB,),
    in_specs=[
        pl.BlockSpec((_LB, 256), lambda i: (i, 0)),
        pl.BlockSpec((_LB, 256), lambda i: (i + LP // _LB, 0)),
        pl.BlockSpec((256, 256), lambda i: (0, 0)),
        pl.BlockSpec((256, 256), lambda i: (0, 0)),
        pl.BlockSpec((1, 256), lambda i: (0, 0)),
        pl.BlockSpec((256, 1), lambda i: (0, 0)),
        pl.BlockSpec(memory_space=pltpu.SMEM),
        pl.BlockSpec(memory_space=pltpu.SMEM),
    ],
    out_specs=pl.BlockSpec((_LB, 1), lambda i: (i, 0)),
    out_shape=jax.ShapeDtypeStruct((LP, 1), F32),
)


def kernel(x, edge_index, edge_attr, all_node_features, rel_features, paths, links,
           x_lin2_W, x_lin2_b, nn_cd_W, nn_cd_b, lstm_Wih, lstm_Whh, lstm_b,
           conv_W, conv_b, lin1_W, lin1_b, lin2_W, lin2_b, prelu_a):
    x_all = jnp.concatenate(
        [all_node_features, rel_features,
         jnp.zeros((VTAB - 12005, NFEAT), F32)], axis=0)
    src = edge_index[0]
    dst = edge_index[1]
    srcp = jnp.concatenate([src, jnp.zeros((EP - E,), I32)]).reshape(NW, -1, 256)
    dum = DUMMY + (jnp.arange(EP - E, dtype=I32) % (SH - NP))
    dstp = jnp.concatenate([dst, dum]).reshape(NW, -1, 256)
    pp = jnp.concatenate([paths, jnp.zeros((PP - P, T), I32)], axis=0)
    pidx = pp.T.reshape(NW, -1, 256)
    zeros64 = jnp.zeros((SH, CONV), F32)
    zeros16 = jnp.zeros((SH, 16), F32)
    ones16 = jnp.zeros((256, 16), F32).at[:, 0].set(1.0)
    a2 = prelu_a.reshape(1, 1)

    cnt = _deg_hist(dstp, zeros16, ones16)
    c0 = cnt[0, :NP]
    c1 = cnt[1, :NP]

    states = []
    for i in range(4):
        feats = _gather_paths(x_all, pidx)
        xwp = _lstm_call(*[feats] * T, lstm_Wih[i], lstm_Whh[i],
                         lstm_b[i].reshape(1, -1), conv_W[i], c0, c1, a2)
        S = _edge_scatter(xwp, srcp, dstp, zeros64)
        xc4 = _fin_call(S[0, :NP], S[1, :NP], xwp, c0, c1, conv_b[i].reshape(1, -1))
        states.append(xc4)
        x_pad = jnp.pad(xc4[:N_NODES], ((0, 0), (0, NFEAT - CONV)))
        x_all = x_all.at[2 * N_NODES:3 * N_NODES].set(x_pad)

    cs4 = jnp.concatenate(states, axis=1)
    cs = cs4[:N_NODES]

    l0 = jnp.pad(links[0], (0, LP - L))
    l1 = jnp.pad(links[1], (0, LP - L)) + 2000
    lidx = jnp.concatenate([l0, l1]).reshape(NW, -1, 128)
    g = _gather_links(cs4, lidx)
    outp = _score_call(g, g, lin1_W[:256], lin1_W[256:], lin1_b.reshape(1, -1),
                       lin2_W, lin2_b.reshape(1, 1), a2)
    out = outp[:L, 0]
    return (out, cs, x_all[:12005])


# speedup vs baseline: 1.5068x; 1.1447x over previous
import functools

import jax
import jax.numpy as jnp
from jax import lax
from jax.experimental import pallas as pl
from jax.experimental.pallas import tpu as pltpu
from jax.experimental.pallas import tpu_sc as plsc

F32 = jnp.float32
I32 = jnp.int32

N_NODES = 4000
NP = 4096
SH = 4224
DUMMY = 4096
NFEAT = 128
HID = 128
CONV = 64
T = 7
P = 8000
PP = 8192
FLAT = PP * T
E = 128000
EP = 131072
L = 10000
LP = 10240
NW = 32

_mesh = lambda: plsc.VectorSubcoreMesh(core_axis_name="c", subcore_axis_name="s")


@functools.cache
def _mk_gather(D, TOT, CH, ring, V, label, stage=True):
    rw = TOT // NW // CH
    depth = max(ring // 2, 1)
    rps = V // 16

    @functools.partial(
        pl.kernel,
        out_type=jax.ShapeDtypeStruct((TOT, D), F32),
        mesh=_mesh(),
        scratch_types=[pltpu.VMEM((rw, CH), I32), pltpu.VMEM((ring * CH, D), F32)]
        + ([pltpu.VMEM_SHARED((V, D), F32)] if stage else [])
        + [pltpu.SemaphoreType.DMA, pltpu.SemaphoreType.DMA],
        compiler_params=pltpu.CompilerParams(use_tc_tiling_on_sc=False),
        name=label,
    )
    def k(table, idx, out, *refs):
        idx_v, rows_v = refs[0], refs[1]
        gsem, wsem = refs[-2], refs[-1]
        c = lax.axis_index("c")
        s = lax.axis_index("s")
        wid = s * 2 + c
        base = wid * rw
        pltpu.sync_copy(idx.at[wid], idx_v)
        if stage:
            tab = refs[2]
            pltpu.sync_copy(table.at[pl.ds(s * rps, rps)], tab.at[pl.ds(s * rps, rps)])
            plsc.subcore_barrier()
        else:
            tab = table

        def buf(j):
            return rows_v.at[pl.ds((j % ring) * CH, CH)]

        gd = {}
        wd = {}
        for j in range(min(depth, rw)):
            gd[j] = pltpu.async_copy(tab.at[idx_v.at[j]], buf(j), gsem)
        for j in range(rw):
            gd[j].wait()
            wd[j] = pltpu.async_copy(buf(j), out.at[pl.ds((base + j) * CH, CH)], wsem)
            if j >= depth:
                wd[j - depth].wait()
            if j + depth < rw:
                gd[j + depth] = pltpu.async_copy(
                    tab.at[idx_v.at[j + depth]], buf(j + depth), gsem)
        for j in range(max(rw - depth, 0), rw):
            wd[j].wait()

    return k


VTAB = 12032


def _gather_paths(table, idx):
    return _mk_gather(CONV, FLAT, 256, 4, VTAB, "path_gather")(table, idx)


def _gather_links(table, idx):
    return _mk_gather(256, 2 * LP, 128, 2, NP, "link_gather", stage=False)(table, idx)


_SROWS = SH // 16


@functools.cache
def _mk_edge_scatter():
    @functools.partial(
        pl.kernel,
        out_type=jax.ShapeDtypeStruct((2, SH, CONV), F32),
        mesh=_mesh(),
        scratch_types=[
            pltpu.VMEM((EP // 256 // NW, 256), I32),
            pltpu.VMEM((EP // 256 // NW, 256), I32),
            pltpu.VMEM((4 * 256, CONV), F32),
            pltpu.VMEM_SHARED((SH, CONV), F32),
            pltpu.VMEM_SHARED((NP, CONV), F32),
            pltpu.SemaphoreType.DMA,
            pltpu.SemaphoreType.DMA,
        ],
        compiler_params=pltpu.CompilerParams(use_tc_tiling_on_sc=False),
        name="edge_scatter",
    )
    def k(xw, sI, dI, z, out, sv, dv, rows, shared, tab, gsem, ssem):
        c = lax.axis_index("c")
        s = lax.axis_index("s")
        wid = s * 2 + c
        nchunk = EP // 256 // NW
        ring, depth = 4, 2
        pltpu.sync_copy(z.at[pl.ds(s * _SROWS, _SROWS)], shared.at[pl.ds(s * _SROWS, _SROWS)])
        pltpu.sync_copy(xw.at[pl.ds(s * (NP // 16), NP // 16)], tab.at[pl.ds(s * (NP // 16), NP // 16)])
        pltpu.sync_copy(sI.at[wid], sv)
        pltpu.sync_copy(dI.at[wid], dv)
        plsc.subcore_barrier()

        def buf(j):
            return rows.at[pl.ds((j % ring) * 256, 256)]

        gd = {}
        sd = {}
        for j in range(depth):
            gd[j] = pltpu.async_copy(tab.at[sv.at[j]], buf(j), gsem)
        for j in range(nchunk):
            gd[j].wait()
            sd[j] = pltpu.async_copy(buf(j), shared.at[dv.at[j]], ssem, add=True)
            if j >= depth:
                sd[j - depth].wait()
            if j + depth < nchunk:
                gd[j + depth] = pltpu.async_copy(
                    tab.at[sv.at[j + depth]], buf(j + depth), gsem)
        for j in range(nchunk - depth, nchunk):
            sd[j].wait()
        plsc.subcore_barrier()
        pltpu.sync_copy(shared.at[pl.ds(s * _SROWS, _SROWS)], out.at[c, pl.ds(s * _SROWS, _SROWS)])

    return k


def _edge_scatter(xw, sI, dI, z):
    return _mk_edge_scatter()(xw, sI, dI, z)


@functools.cache
def _mk_deg_hist():
    @functools.partial(
        pl.kernel,
        out_type=jax.ShapeDtypeStruct((2, SH, 16), F32),
        mesh=_mesh(),
        scratch_types=[
            pltpu.VMEM((EP // 256 // NW, 256), I32),
            pltpu.VMEM((256, 16), F32),
            pltpu.VMEM_SHARED((SH, 16), F32),
            pltpu.SemaphoreType.DMA,
        ],
        compiler_params=pltpu.CompilerParams(use_tc_tiling_on_sc=False),
        name="deg_hist",
    )
    def k(dI, z, ones, out, dv, ones_v, shared, sem):
        c = lax.axis_index("c")
        s = lax.axis_index("s")
        wid = s * 2 + c
        nchunk = EP // 256 // NW
        pltpu.sync_copy(z.at[pl.ds(s * _SROWS, _SROWS)], shared.at[pl.ds(s * _SROWS, _SROWS)])
        pltpu.sync_copy(dI.at[wid], dv)
        pltpu.sync_copy(ones, ones_v)
        plsc.subcore_barrier()

        sd = {}
        for j in range(nchunk):
            sd[j] = pltpu.async_copy(ones_v, shared.at[dv.at[j]], sem, add=True)
            if j >= 8:
                sd[j - 8].wait()
        for j in range(nchunk - 8, nchunk):
            sd[j].wait()
        plsc.subcore_barrier()
        pltpu.sync_copy(shared.at[pl.ds(s * _SROWS, _SROWS)], out.at[c, pl.ds(s * _SROWS, _SROWS)])

    return k


def _deg_hist(dI, z, ones):
    return _mk_deg_hist()(dI, z, ones)


_PB = 512
_NB = 256
_GRID = PP // _PB


def _lstm_body(a0, a1, a2, a3, a4, a5, a6, b0, b1, b2, b3, b4, b5, b6,
               wihA, wihB, whh, b, cw, c0, c1, a, out):
    fA = (a0, a1, a2, a3, a4, a5, a6)
    fB = (b0, b1, b2, b3, b4, b5, b6)
    BF = jnp.bfloat16
    W_ihA = wihA[:].astype(BF)
    W_ihB = wihB[:].astype(BF)
    W_hh = whh[:].astype(BF)
    bb = b[:]
    h = jnp.zeros((_PB, HID), F32)
    c = jnp.zeros((_PB, HID), F32)
    for t in range(T):
        g = (jnp.dot(fA[t][:].astype(BF), W_ihA, preferred_element_type=F32)
             + jnp.dot(fB[t][:].astype(BF), W_ihB, preferred_element_type=F32)
             + jnp.dot(h.astype(BF), W_hh, preferred_element_type=F32) + bb)
        ig = jax.nn.sigmoid(g[:, :HID])
        fg = jax.nn.sigmoid(g[:, HID:2 * HID])
        gg = jnp.tanh(g[:, 2 * HID:3 * HID])
        og = jax.nn.sigmoid(g[:, 3 * HID:])
        c = fg * c + ig * gg
        h = og * jnp.tanh(c)
    hr = h.reshape(_NB, 2 * HID)
    hm = (hr[:, :HID] + hr[:, HID:]) * 0.5
    av = a[0, 0]
    hp = jnp.where(hm > 0, hm, av * hm)
    xw = jnp.dot(hp, cw[:], preferred_element_type=F32)
    dinv = lax.rsqrt(c0[:, :1] + c1[:, :1] + 1.0)
    out[:] = xw * dinv


_lstm_call = pl.pallas_call(
    _lstm_body,
    grid=(_GRID,),
    in_specs=[pl.BlockSpec((_PB, CONV), functools.partial(lambda i, t: (t * _GRID + i, 0), t=t))
              for t in range(T)] * 2
    + [
        pl.BlockSpec((CONV, 4 * HID), lambda i: (0, 0)),
        pl.BlockSpec((CONV, 4 * HID), lambda i: (0, 0)),
        pl.BlockSpec((HID, 4 * HID), lambda i: (0, 0)),
        pl.BlockSpec((1, 4 * HID), lambda i: (0, 0)),
        pl.BlockSpec((HID, CONV), lambda i: (0, 0)),
        pl.BlockSpec((_NB, 16), lambda i: (i, 0)),
        pl.BlockSpec((_NB, 16), lambda i: (i, 0)),
        pl.BlockSpec(memory_space=pltpu.SMEM),
    ],
    out_specs=pl.BlockSpec((_NB, CONV), lambda i: (i, 0)),
    out_shape=jax.ShapeDtypeStruct((NP, CONV), F32),
)


def _fin_body(s0, s1, xw, c0, c1, b, out):
    acc = s0[:] + s1[:] + xw[:]
    dinv = lax.rsqrt(c0[:, :1] + c1[:, :1] + 1.0)
    v = acc * dinv + b[:]
    n = jnp.sqrt(jnp.sum(v * v, axis=1, keepdims=True))
    out[:] = v / jnp.maximum(n, 1e-12)


_fin_call = pl.pallas_call(
    _fin_body,
    grid=(4,),
    in_specs=[
        pl.BlockSpec((1024, CONV), lambda i: (i, 0)),
        pl.BlockSpec((1024, CONV), lambda i: (i, 0)),
        pl.BlockSpec((1024, CONV), lambda i: (i, 0)),
        pl.BlockSpec((1024, 16), lambda i: (i, 0)),
        pl.BlockSpec((1024, 16), lambda i: (i, 0)),
        pl.BlockSpec((1, CONV), lambda i: (0, 0)),
    ],
    out_specs=pl.BlockSpec((1024, CONV), lambda i: (i, 0)),
    out_shape=jax.ShapeDtypeStruct((NP, CONV), F32),
)


_LB = 512


def _score_body(ga, gb, w1a, w1b, b1, w2, b2, a, out):
    BF = jnp.bfloat16
    h = (jnp.dot(ga[:].astype(BF), w1a[:].astype(BF), preferred_element_type=F32)
         + jnp.dot(gb[:].astype(BF), w1b[:].astype(BF), preferred_element_type=F32) + b1[:])
    av = a[0, 0]
    h = jnp.where(h > 0, h, av * h)
    out[:] = jnp.dot(h, w2[:], preferred_element_type=F32) + b2[0, 0]


_score_call = pl.pallas_call(
    _score_body,
    grid=(LP // _LB,),
    in_specs=[
        pl.BlockSpec((_LB, 256), lambda i: (i, 0)),
        pl.BlockSpec((_LB, 256), lambda i: (i + LP // _LB, 0)),
        pl.BlockSpec((256, 256), lambda i: (0, 0)),
        pl.BlockSpec((256, 256), lambda i: (0, 0)),
        pl.BlockSpec((1, 256), lambda i: (0, 0)),
        pl.BlockSpec((256, 1), lambda i: (0, 0)),
        pl.BlockSpec(memory_space=pltpu.SMEM),
        pl.BlockSpec(memory_space=pltpu.SMEM),
    ],
    out_specs=pl.BlockSpec((_LB, 1), lambda i: (i, 0)),
    out_shape=jax.ShapeDtypeStruct((LP, 1), F32),
)


def kernel(x, edge_index, edge_attr, all_node_features, rel_features, paths, links,
           x_lin2_W, x_lin2_b, nn_cd_W, nn_cd_b, lstm_Wih, lstm_Whh, lstm_b,
           conv_W, conv_b, lin1_W, lin1_b, lin2_W, lin2_b, prelu_a):
    x_all0 = jnp.concatenate(
        [all_node_features, rel_features,
         jnp.zeros((VTAB - 12005, NFEAT), F32)], axis=0)
    tabA = x_all0[:, :CONV]
    tabB1 = x_all0[:, CONV:]
    tabBs = tabB1.at[2 * N_NODES:3 * N_NODES].set(jnp.zeros((N_NODES, CONV), F32))
    src = edge_index[0]
    dst = edge_index[1]
    srcp = jnp.concatenate([src, jnp.zeros((EP - E,), I32)]).reshape(NW, -1, 256)
    dum = DUMMY + (jnp.arange(EP - E, dtype=I32) % (SH - NP))
    dstp = jnp.concatenate([dst, dum]).reshape(NW, -1, 256)
    pp = jnp.concatenate([paths, jnp.zeros((PP - P, T), I32)], axis=0)
    pidx = pp.T.reshape(NW, -1, 256)
    zeros64 = jnp.zeros((SH, CONV), F32)
    zeros16 = jnp.zeros((SH, 16), F32)
    ones16 = jnp.zeros((256, 16), F32).at[:, 0].set(1.0)
    a2 = prelu_a.reshape(1, 1)

    cnt = _deg_hist(dstp, zeros16, ones16)
    c0 = cnt[0, :NP]
    c1 = cnt[1, :NP]

    ftB1 = _gather_paths(tabB1, pidx)
    ftBs = _gather_paths(tabBs, pidx)
    states = []
    for i in range(4):
        ftA = _gather_paths(tabA, pidx)
        ftB = ftB1 if i == 0 else ftBs
        xwp = _lstm_call(*[ftA] * T, *[ftB] * T, lstm_Wih[i][:CONV], lstm_Wih[i][CONV:],
                         lstm_Whh[i], lstm_b[i].reshape(1, -1), conv_W[i], c0, c1, a2)
        S = _edge_scatter(xwp, srcp, dstp, zeros64)
        xc4 = _fin_call(S[0, :NP], S[1, :NP], xwp, c0, c1, conv_b[i].reshape(1, -1))
        states.append(xc4)
        if i < 3:
            tabA = tabA.at[2 * N_NODES:3 * N_NODES].set(xc4[:N_NODES])

    x_all = jnp.concatenate(
        [jnp.concatenate([tabA[:2 * N_NODES], states[3][:N_NODES],
                          tabA[3 * N_NODES:12005]], axis=0),
         tabBs[:12005]], axis=1)
    cs4 = jnp.concatenate(states, axis=1)
    cs = cs4[:N_NODES]

    l0 = jnp.pad(links[0], (0, LP - L))
    l1 = jnp.pad(links[1], (0, LP - L)) + 2000
    lidx = jnp.concatenate([l0, l1]).reshape(NW, -1, 128)
    g = _gather_links(cs4, lidx)
    outp = _score_call(g, g, lin1_W[:256], lin1_W[256:], lin1_b.reshape(1, -1),
                       lin2_W, lin2_b.reshape(1, 1), a2)
    out = outp[:L, 0]
    return (out, cs, x_all)


# speedup vs baseline: 1.6693x; 1.1079x over previous
import functools

import jax
import jax.numpy as jnp
from jax import lax
from jax.experimental import pallas as pl
from jax.experimental.pallas import tpu as pltpu
from jax.experimental.pallas import tpu_sc as plsc

F32 = jnp.float32
I32 = jnp.int32

N_NODES = 4000
NP = 4096
SH = 4224
DUMMY = 4096
NFEAT = 128
HID = 128
CONV = 64
T = 7
P = 8000
PP = 8192
FLAT = PP * T
E = 128000
EP = 131072
L = 10000
LP = 10240
NW = 32

_mesh = lambda: plsc.VectorSubcoreMesh(core_axis_name="c", subcore_axis_name="s")


@functools.cache
def _mk_gather(D, TOT, CH, ring, V, label, stage=True):
    rw = TOT // NW // CH
    depth = max(ring // 2, 1)
    rps = V // 16

    @functools.partial(
        pl.kernel,
        out_type=jax.ShapeDtypeStruct((TOT, D), F32),
        mesh=_mesh(),
        scratch_types=[pltpu.VMEM((rw, CH), I32), pltpu.VMEM((ring * CH, D), F32)]
        + ([pltpu.VMEM_SHARED((V, D), F32)] if stage else [])
        + [pltpu.SemaphoreType.DMA, pltpu.SemaphoreType.DMA],
        compiler_params=pltpu.CompilerParams(use_tc_tiling_on_sc=False),
        name=label,
    )
    def k(table, idx, out, *refs):
        idx_v, rows_v = refs[0], refs[1]
        gsem, wsem = refs[-2], refs[-1]
        c = lax.axis_index("c")
        s = lax.axis_index("s")
        wid = s * 2 + c
        base = wid * rw
        pltpu.sync_copy(idx.at[wid], idx_v)
        if stage:
            tab = refs[2]
            pltpu.sync_copy(table.at[pl.ds(s * rps, rps)], tab.at[pl.ds(s * rps, rps)])
            plsc.subcore_barrier()
        else:
            tab = table

        def buf(j):
            return rows_v.at[pl.ds((j % ring) * CH, CH)]

        gd = {}
        wd = {}
        for j in range(min(depth, rw)):
            gd[j] = pltpu.async_copy(tab.at[idx_v.at[j]], buf(j), gsem)
        for j in range(rw):
            gd[j].wait()
            wd[j] = pltpu.async_copy(buf(j), out.at[pl.ds((base + j) * CH, CH)], wsem)
            if j >= depth:
                wd[j - depth].wait()
            if j + depth < rw:
                gd[j + depth] = pltpu.async_copy(
                    tab.at[idx_v.at[j + depth]], buf(j + depth), gsem)
        for j in range(max(rw - depth, 0), rw):
            wd[j].wait()

    return k


VTAB = 12032


@functools.cache
def _mk_gather_adup():
    CH = 256
    ring = 2
    rw = FLAT // NW // CH
    rps = VTAB // 16

    @functools.partial(
        pl.kernel,
        out_type=jax.ShapeDtypeStruct((FLAT, NFEAT), F32),
        mesh=_mesh(),
        scratch_types=[
            pltpu.VMEM((rw, CH), I32),
            pltpu.VMEM((ring * CH, CONV), F32),
            pltpu.VMEM_SHARED((VTAB, CONV), F32),
            pltpu.SemaphoreType.DMA,
            pltpu.SemaphoreType.DMA,
        ],
        compiler_params=pltpu.CompilerParams(use_tc_tiling_on_sc=False),
        name="path_gather_a",
    )
    def k(tabA, idx, out, idx_v, bufA, shA, gsem, wsem):
        c = lax.axis_index("c")
        s = lax.axis_index("s")
        wid = s * 2 + c
        base = wid * rw
        pltpu.sync_copy(idx.at[wid], idx_v)
        pltpu.sync_copy(tabA.at[pl.ds(s * rps, rps)], shA.at[pl.ds(s * rps, rps)])
        plsc.subcore_barrier()

        def bA(j):
            return bufA.at[pl.ds((j % ring) * CH, CH)]

        gd = {}
        wd = {}
        gd[0] = pltpu.async_copy(shA.at[idx_v.at[0]], bA(0), gsem)
        for j in range(rw):
            gd[j].wait()
            wd[j] = (
                pltpu.async_copy(bA(j), out.at[pl.ds((base + j) * CH, CH), pl.ds(0, CONV)], wsem),
                pltpu.async_copy(bA(j), out.at[pl.ds((base + j) * CH, CH), pl.ds(CONV, CONV)], wsem),
            )
            if j >= 1:
                wd[j - 1][0].wait()
                wd[j - 1][1].wait()
            if j + 1 < rw:
                gd[j + 1] = pltpu.async_copy(shA.at[idx_v.at[j + 1]], bA(j + 1), gsem)
        wd[rw - 1][0].wait()
        wd[rw - 1][1].wait()

    return k


def _gather_paths_a(tabA, idx):
    return _mk_gather_adup()(tabA, idx)


def _gather_paths_b(tabB, idx):
    return _mk_gather(CONV, FLAT, 256, 4, VTAB, "path_gather_b")(tabB, idx)


def _gather_links(table, idx):
    return _mk_gather(256, 2 * LP, 128, 2, NP, "link_gather", stage=False)(table, idx)


_SROWS = SH // 16


@functools.cache
def _mk_edge_scatter():
    @functools.partial(
        pl.kernel,
        out_type=jax.ShapeDtypeStruct((2, SH, CONV), F32),
        mesh=_mesh(),
        scratch_types=[
            pltpu.VMEM((EP // 256 // NW, 256), I32),
            pltpu.VMEM((EP // 256 // NW, 256), I32),
            pltpu.VMEM((4 * 256, CONV), F32),
            pltpu.VMEM_SHARED((SH, CONV), F32),
            pltpu.VMEM_SHARED((NP, CONV), F32),
            pltpu.SemaphoreType.DMA,
            pltpu.SemaphoreType.DMA,
        ],
        compiler_params=pltpu.CompilerParams(use_tc_tiling_on_sc=False),
        name="edge_scatter",
    )
    def k(xw, sI, dI, z, out, sv, dv, rows, shared, tab, gsem, ssem):
        c = lax.axis_index("c")
        s = lax.axis_index("s")
        wid = s * 2 + c
        nchunk = EP // 256 // NW
        ring, depth = 4, 2
        pltpu.sync_copy(z.at[pl.ds(s * _SROWS, _SROWS)], shared.at[pl.ds(s * _SROWS, _SROWS)])
        pltpu.sync_copy(xw.at[pl.ds(s * (NP // 16), NP // 16)], tab.at[pl.ds(s * (NP // 16), NP // 16)])
        pltpu.sync_copy(sI.at[wid], sv)
        pltpu.sync_copy(dI.at[wid], dv)
        plsc.subcore_barrier()

        def buf(j):
            return rows.at[pl.ds((j % ring) * 256, 256)]

        gd = {}
        sd = {}
        for j in range(depth):
            gd[j] = pltpu.async_copy(tab.at[sv.at[j]], buf(j), gsem)
        for j in range(nchunk):
            gd[j].wait()
            sd[j] = pltpu.async_copy(buf(j), shared.at[dv.at[j]], ssem, add=True)
            if j >= depth:
                sd[j - depth].wait()
            if j + depth < nchunk:
                gd[j + depth] = pltpu.async_copy(
                    tab.at[sv.at[j + depth]], buf(j + depth), gsem)
        for j in range(nchunk - depth, nchunk):
            sd[j].wait()
        plsc.subcore_barrier()
        pltpu.sync_copy(shared.at[pl.ds(s * _SROWS, _SROWS)], out.at[c, pl.ds(s * _SROWS, _SROWS)])

    return k


def _edge_scatter(xw, sI, dI, z):
    return _mk_edge_scatter()(xw, sI, dI, z)


@functools.cache
def _mk_deg_hist():
    @functools.partial(
        pl.kernel,
        out_type=jax.ShapeDtypeStruct((2, SH, 16), F32),
        mesh=_mesh(),
        scratch_types=[
            pltpu.VMEM((EP // 256 // NW, 256), I32),
            pltpu.VMEM((256, 16), F32),
            pltpu.VMEM_SHARED((SH, 16), F32),
            pltpu.SemaphoreType.DMA,
        ],
        compiler_params=pltpu.CompilerParams(use_tc_tiling_on_sc=False),
        name="deg_hist",
    )
    def k(dI, z, ones, out, dv, ones_v, shared, sem):
        c = lax.axis_index("c")
        s = lax.axis_index("s")
        wid = s * 2 + c
        nchunk = EP // 256 // NW
        pltpu.sync_copy(z.at[pl.ds(s * _SROWS, _SROWS)], shared.at[pl.ds(s * _SROWS, _SROWS)])
        pltpu.sync_copy(dI.at[wid], dv)
        pltpu.sync_copy(ones, ones_v)
        plsc.subcore_barrier()

        sd = {}
        for j in range(nchunk):
            sd[j] = pltpu.async_copy(ones_v, shared.at[dv.at[j]], sem, add=True)
            if j >= 8:
                sd[j - 8].wait()
        for j in range(nchunk - 8, nchunk):
            sd[j].wait()
        plsc.subcore_barrier()
        pltpu.sync_copy(shared.at[pl.ds(s * _SROWS, _SROWS)], out.at[c, pl.ds(s * _SROWS, _SROWS)])

    return k


def _deg_hist(dI, z, ones):
    return _mk_deg_hist()(dI, z, ones)


_PB = 512
_NB = 256
_GRID = PP // _PB


def _lstm_body(f0, f1, f2, f3, f4, f5, f6, g0, g1, g2, g3, g4, g5, g6,
               wpad, wihB, whh, b, cw, c0, c1, a, out):
    fs = (f0, f1, f2, f3, f4, f5, f6)
    gs = (g0, g1, g2, g3, g4, g5, g6)
    BF = jnp.bfloat16
    W_pad = wpad[:].astype(BF)
    W_ihB = wihB[:].astype(BF)
    W_hh = whh[:].astype(BF)
    bb = b[:]
    h = jnp.zeros((_PB, HID), F32)
    c = jnp.zeros((_PB, HID), F32)
    for t in range(T):
        g = (jnp.dot(fs[t][:].astype(BF), W_pad, preferred_element_type=F32)
             + jnp.dot(gs[t][:].astype(BF), W_ihB, preferred_element_type=F32)
             + jnp.dot(h.astype(BF), W_hh, preferred_element_type=F32) + bb)
        ig = jax.nn.sigmoid(g[:, :HID])
        fg = jax.nn.sigmoid(g[:, HID:2 * HID])
        gg = jnp.tanh(g[:, 2 * HID:3 * HID])
        og = jax.nn.sigmoid(g[:, 3 * HID:])
        c = fg * c + ig * gg
        h = og * jnp.tanh(c)
    hr = h.reshape(_NB, 2 * HID)
    hm = (hr[:, :HID] + hr[:, HID:]) * 0.5
    av = a[0, 0]
    hp = jnp.where(hm > 0, hm, av * hm)
    xw = jnp.dot(hp, cw[:], preferred_element_type=F32)
    dinv = lax.rsqrt(c0[:, :1] + c1[:, :1] + 1.0)
    out[:] = xw * dinv


_lstm_call = pl.pallas_call(
    _lstm_body,
    grid=(_GRID,),
    in_specs=[pl.BlockSpec((_PB, NFEAT), functools.partial(lambda i, t: (t * _GRID + i, 0), t=t))
              for t in range(T)]
    + [pl.BlockSpec((_PB, CONV), functools.partial(lambda i, t: (t * _GRID + i, 0), t=t))
       for t in range(T)]
    + [
        pl.BlockSpec((NFEAT, 4 * HID), lambda i: (0, 0)),
        pl.BlockSpec((CONV, 4 * HID), lambda i: (0, 0)),
        pl.BlockSpec((HID, 4 * HID), lambda i: (0, 0)),
        pl.BlockSpec((1, 4 * HID), lambda i: (0, 0)),
        pl.BlockSpec((HID, CONV), lambda i: (0, 0)),
        pl.BlockSpec((_NB, 16), lambda i: (i, 0)),
        pl.BlockSpec((_NB, 16), lambda i: (i, 0)),
        pl.BlockSpec(memory_space=pltpu.SMEM),
    ],
    out_specs=pl.BlockSpec((_NB, CONV), lambda i: (i, 0)),
    out_shape=jax.ShapeDtypeStruct((NP, CONV), F32),
)


def _fin_body(s0, s1, xw, c0, c1, b, out):
    acc = s0[:] + s1[:] + xw[:]
    dinv = lax.rsqrt(c0[:, :1] + c1[:, :1] + 1.0)
    v = acc * dinv + b[:]
    n = jnp.sqrt(jnp.sum(v * v, axis=1, keepdims=True))
    out[:] = v / jnp.maximum(n, 1e-12)


_fin_call = pl.pallas_call(
    _fin_body,
    grid=(4,),
    in_specs=[
        pl.BlockSpec((1024, CONV), lambda i: (i, 0)),
        pl.BlockSpec((1024, CONV), lambda i: (i, 0)),
        pl.BlockSpec((1024, CONV), lambda i: (i, 0)),
        pl.BlockSpec((1024, 16), lambda i: (i, 0)),
        pl.BlockSpec((1024, 16), lambda i: (i, 0)),
        pl.BlockSpec((1, CONV), lambda i: (0, 0)),
    ],
    out_specs=pl.BlockSpec((1024, CONV), lambda i: (i, 0)),
    out_shape=jax.ShapeDtypeStruct((NP, CONV), F32),
)


_LB = 512


def _score_body(ga, gb, w1a, w1b, b1, w2, b2, a, out):
    BF = jnp.bfloat16
    h = (jnp.dot(ga[:].astype(BF), w1a[:].astype(BF), preferred_element_type=F32)
         + jnp.dot(gb[:].astype(BF), w1b[:].astype(BF), preferred_element_type=F32) + b1[:])
    av = a[0, 0]
    h = jnp.where(h > 0, h, av * h)
    out[:] = jnp.dot(h, w2[:], preferred_element_type=F32) + b2[0, 0]


_score_call = pl.pallas_call(
    _score_body,
    grid=(LP // _LB,),
    in_specs=[
        pl.BlockSpec((_LB, 256), lambda i: (i, 0)),
        pl.BlockSpec((_LB, 256), lambda i: (i + LP // _LB, 0)),
        pl.BlockSpec((256, 256), lambda i: (0, 0)),
        pl.BlockSpec((256, 256), lambda i: (0, 0)),
        pl.BlockSpec((1, 256), lambda i: (0, 0)),
        pl.BlockSpec((256, 1), lambda i: (0, 0)),
        pl.BlockSpec(memory_space=pltpu.SMEM),
        pl.BlockSpec(memory_space=pltpu.SMEM),
    ],
    out_specs=pl.BlockSpec((_LB, 1), lambda i: (i, 0)),
    out_shape=jax.ShapeDtypeStruct((LP, 1), F32),
)


def kernel(x, edge_index, edge_attr, all_node_features, rel_features, paths, links,
           x_lin2_W, x_lin2_b, nn_cd_W, nn_cd_b, lstm_Wih, lstm_Whh, lstm_b,
           conv_W, conv_b, lin1_W, lin1_b, lin2_W, lin2_b, prelu_a):
    x_all0 = jnp.concatenate(
        [all_node_features, rel_features,
         jnp.zeros((VTAB - 12005, NFEAT), F32)], axis=0)
    tabA = x_all0[:, :CONV]
    tabB1 = x_all0[:, CONV:]
    tabBs = tabB1.at[2 * N_NODES:3 * N_NODES].set(jnp.zeros((N_NODES, CONV), F32))
    src = edge_index[0]
    dst = edge_index[1]
    srcp = jnp.concatenate([src, jnp.zeros((EP - E,), I32)]).reshape(NW, -1, 256)
    dum = DUMMY + (jnp.arange(EP - E, dtype=I32) % (SH - NP))
    dstp = jnp.concatenate([dst, dum]).reshape(NW, -1, 256)
    pp = jnp.concatenate([paths, jnp.zeros((PP - P, T), I32)], axis=0)
    pidx = pp.T.reshape(NW, -1, 256)
    zeros64 = jnp.zeros((SH, CONV), F32)
    zeros16 = jnp.zeros((SH, 16), F32)
    ones16 = jnp.zeros((256, 16), F32).at[:, 0].set(1.0)
    a2 = prelu_a.reshape(1, 1)

    cnt = _deg_hist(dstp, zeros16, ones16)
    c0 = cnt[0, :NP]
    c1 = cnt[1, :NP]

    ftB1 = _gather_paths_b(tabB1, pidx)
    ftBs = _gather_paths_b(tabBs, pidx)
    zpad = jnp.zeros((CONV, 4 * HID), F32)
    states = []
    for i in range(4):
        ftA = _gather_paths_a(tabA, pidx)
        ftB = ftB1 if i == 0 else ftBs
        wpad = jnp.concatenate([lstm_Wih[i][:CONV], zpad], axis=0)
        xwp = _lstm_call(*[ftA] * T, *[ftB] * T, wpad, lstm_Wih[i][CONV:],
                         lstm_Whh[i], lstm_b[i].reshape(1, -1), conv_W[i], c0, c1, a2)
        S = _edge_scatter(xwp, srcp, dstp, zeros64)
        xc4 = _fin_call(S[0, :NP], S[1, :NP], xwp, c0, c1, conv_b[i].reshape(1, -1))
        states.append(xc4)
        if i < 3:
            tabA = tabA.at[2 * N_NODES:3 * N_NODES].set(xc4[:N_NODES])

    x_all = jnp.concatenate(
        [jnp.concatenate([tabA[:2 * N_NODES], states[3][:N_NODES],
                          tabA[3 * N_NODES:12005]], axis=0),
         tabBs[:12005]], axis=1)
    cs4 = jnp.concatenate(states, axis=1)
    cs = cs4[:N_NODES]

    l0 = jnp.pad(links[0], (0, LP - L))
    l1 = jnp.pad(links[1], (0, LP - L)) + 2000
    lidx = jnp.concatenate([l0, l1]).reshape(NW, -1, 128)
    g = _gather_links(cs4, lidx)
    outp = _score_call(g, g, lin1_W[:256], lin1_W[256:], lin1_b.reshape(1, -1),
                       lin2_W, lin2_b.reshape(1, 1), a2)
    out = outp[:L, 0]
    return (out, cs, x_all)
